# Initial kernel scaffold; baseline (speedup 1.0000x reference)
import jax
import jax.numpy as jnp
from jax.experimental import pallas as pl


def kernel(x, edge_index, W0, b0, W1, b1, W2, b2):
    raise NotImplementedError("write your pallas kernel here")


# speedup vs baseline: 8.4627x; 8.4627x over previous
import functools

import jax
import jax.numpy as jnp
from jax import lax
from jax.experimental import pallas as pl
from jax.experimental.pallas import tpu as pltpu
from jax.experimental.pallas import tpu_sc as plsc

N = 10000
E = 320000
D = 128

NP = 10240
NC = 2
NS = 16
NW = NC * NS
CHUNK = 128
NCH = 80
EPAD = NW * NCH * CHUNK
RPT = NP // NS
ZR = 16
BLK = 512

_mesh = plsc.VectorSubcoreMesh(
    core_axis_name="c", subcore_axis_name="s", num_cores=NC, num_subcores=NS)


def _deg_body(dst_hbm, out_hbm, dstidx_v, ones_v, zer_v, deg_sh):
    c = lax.axis_index("c")
    s = lax.axis_index("s")
    wid = c * NS + s

    for j in range(CHUNK // 16):
        ones_v[pl.ds(j * 16, 16)] = jnp.full((16,), 1.0, jnp.float32)

    def zrow(r, _):
        zer_v[pl.ds(r * 16, 16)] = jnp.zeros((16,), jnp.float32)
        return 0
    lax.fori_loop(0, RPT // 16, zrow, 0)
    pltpu.sync_copy(zer_v, deg_sh.at[pl.ds(s * RPT, RPT)])
    plsc.subcore_barrier()

    pltpu.sync_copy(dst_hbm.at[pl.ds(wid * NCH, NCH)], dstidx_v)

    def step(j, _):
        pltpu.sync_copy(ones_v, deg_sh.at[dstidx_v.at[j]], add=True)
        return 0
    lax.fori_loop(0, NCH, step, 0)
    plsc.subcore_barrier()

    pltpu.sync_copy(deg_sh.at[pl.ds(s * RPT, RPT)],
                    out_hbm.at[pl.ds(c * NP + s * RPT, RPT)])


_deg_call = functools.partial(
    pl.kernel, _deg_body,
    out_type=jax.ShapeDtypeStruct((NC * NP,), jnp.float32),
    mesh=_mesh,
    scratch_types=[
        pltpu.VMEM((NCH, CHUNK), jnp.int32),
        pltpu.VMEM((CHUNK,), jnp.float32),
        pltpu.VMEM((RPT,), jnp.float32),
        pltpu.VMEM_SHARED((NP,), jnp.float32),
    ],
)()


def _msg_body(g_hbm, src_hbm, dst_hbm, out_hbm,
              srcidx_v, dstidx_v, rows_v, zbuf_v, agg_sh, sem):
    c = lax.axis_index("c")
    s = lax.axis_index("s")
    wid = c * NS + s

    for r in range(ZR):
        for j in range(D // 16):
            zbuf_v[r, pl.ds(j * 16, 16)] = jnp.zeros((16,), jnp.float32)

    def zcp(j, _):
        pltpu.sync_copy(zbuf_v, agg_sh.at[pl.ds(s * RPT + j * ZR, ZR)])
        return 0
    lax.fori_loop(0, RPT // ZR, zcp, 0)
    plsc.subcore_barrier()

    pltpu.sync_copy(src_hbm.at[pl.ds(wid * NCH, NCH)], srcidx_v)
    pltpu.sync_copy(dst_hbm.at[pl.ds(wid * NCH, NCH)], dstidx_v)

    def step(j, _):
        pltpu.async_copy(g_hbm.at[srcidx_v.at[j]], rows_v, sem).wait()
        pltpu.sync_copy(rows_v, agg_sh.at[dstidx_v.at[j]], add=True)
        return 0
    lax.fori_loop(0, NCH, step, 0)
    plsc.subcore_barrier()

    pltpu.sync_copy(agg_sh.at[pl.ds(s * RPT, RPT)],
                    out_hbm.at[c].at[pl.ds(s * RPT, RPT)])


_msg_call = functools.partial(
    pl.kernel, _msg_body,
    out_type=jax.ShapeDtypeStruct((NC, NP, D), jnp.float32),
    mesh=_mesh,
    scratch_types=[
        pltpu.VMEM((NCH, CHUNK), jnp.int32),
        pltpu.VMEM((NCH, CHUNK), jnp.int32),
        pltpu.VMEM((CHUNK, D), jnp.float32),
        pltpu.VMEM((ZR, D), jnp.float32),
        pltpu.VMEM_SHARED((NP, D), jnp.float32),
        pltpu.SemaphoreType.DMA,
    ],
)()


def _dis_body(dp_ref, o_ref):
    dsum = dp_ref[0] + dp_ref[1] + 1.0
    o_ref[...] = lax.rsqrt(jnp.maximum(dsum, 1.0))


def _dis_call(deg_parts):
    return pl.pallas_call(
        _dis_body,
        out_shape=jax.ShapeDtypeStruct((NP // D, D), jnp.float32),
    )(deg_parts)


def _mm_body(z_ref, w_ref, dis_ref, o_ref):
    acc = jnp.dot(z_ref[...], w_ref[...], preferred_element_type=jnp.float32)
    o_ref[...] = acc * dis_ref[...]


def _mm_scale(z, w, dis):
    return pl.pallas_call(
        _mm_body,
        grid=(NP // BLK,),
        in_specs=[
            pl.BlockSpec((BLK, D), lambda i: (i, 0)),
            pl.BlockSpec((D, D), lambda i: (0, 0)),
            pl.BlockSpec((BLK, 1), lambda i: (i, 0)),
        ],
        out_specs=pl.BlockSpec((BLK, D), lambda i: (i, 0)),
        out_shape=jax.ShapeDtypeStruct((NP, D), jnp.float32),
    )(z, w, dis)


def _comb_body_plain(p_ref, g_ref, dis_ref, b_ref, o_ref, *, relu):
    v = (p_ref[0] + p_ref[1] + g_ref[...]) * dis_ref[...] + b_ref[...]
    o_ref[...] = jnp.maximum(v, 0.0) if relu else v


def _comb_body_skip(p_ref, g_ref, dis_ref, b_ref, skip_ref, o_ref):
    v = (p_ref[0] + p_ref[1] + g_ref[...]) * dis_ref[...] + b_ref[...]
    o_ref[...] = jnp.maximum(skip_ref[...] + v, 0.0)


def _combine(parts, g, dis, b, skip=None, relu=False):
    in_specs = [
        pl.BlockSpec((NC, BLK, D), lambda i: (0, i, 0)),
        pl.BlockSpec((BLK, D), lambda i: (i, 0)),
        pl.BlockSpec((BLK, 1), lambda i: (i, 0)),
        pl.BlockSpec((1, D), lambda i: (0, 0)),
    ]
    args = [parts, g, dis, b.reshape(1, D)]
    if skip is None:
        body = functools.partial(_comb_body_plain, relu=relu)
    else:
        body = _comb_body_skip
        in_specs.append(pl.BlockSpec((BLK, D), lambda i: (i, 0)))
        args.append(skip)
    return pl.pallas_call(
        body,
        grid=(NP // BLK,),
        in_specs=in_specs,
        out_specs=pl.BlockSpec((BLK, D), lambda i: (i, 0)),
        out_shape=jax.ShapeDtypeStruct((NP, D), jnp.float32),
    )(*args)


def kernel(x, edge_index, W0, b0, W1, b1, W2, b2):
    src = edge_index[0]
    dst = edge_index[1]
    pad_e = EPAD - E
    src_p = jnp.concatenate(
        [src, jnp.zeros((pad_e,), jnp.int32)]).reshape(NW * NCH, CHUNK)
    dst_p = jnp.concatenate(
        [dst, jnp.full((pad_e,), N, jnp.int32)]).reshape(NW * NCH, CHUNK)
    x_p = jnp.concatenate([x, jnp.zeros((NP - N, D), x.dtype)])

    deg_parts = _deg_call(dst_p)
    dis = _dis_call(deg_parts.reshape(NC, NP // D, D)).reshape(NP, 1)

    g0 = _mm_scale(x_p, W0, dis)
    p0 = _msg_call(g0, src_p, dst_p)
    a0 = _combine(p0, g0, dis, b0, relu=True)

    g1 = _mm_scale(a0, W1, dis)
    p1 = _msg_call(g1, src_p, dst_p)
    a1 = _combine(p1, g1, dis, b1, skip=a0)

    g2 = _mm_scale(a1, W2, dis)
    p2 = _msg_call(g2, src_p, dst_p)
    out = _combine(p2, g2, dis, b2, relu=False)
    return out[:N]


# speedup vs baseline: 9.9447x; 1.1751x over previous
import functools

import jax
import jax.numpy as jnp
from jax import lax
from jax.experimental import pallas as pl
from jax.experimental.pallas import tpu as pltpu
from jax.experimental.pallas import tpu_sc as plsc

N = 10000
E = 320000
D = 128

NP = 10240
NC = 2
NS = 16
NW = NC * NS
CHUNK = 128
NCH = 80
QCH = 16
NQ = NCH // QCH
EPAD = NW * NCH * CHUNK
RPT = NP // NS
NBUF = 2
BLK = 512

_mesh = plsc.VectorSubcoreMesh(
    core_axis_name="c", subcore_axis_name="s", num_cores=NC, num_subcores=NS)


def _deg_body(dst_hbm, out_hbm, dstidx_v, ones_v, zer_v, deg_sh):
    c = lax.axis_index("c")
    s = lax.axis_index("s")
    wid = c * NS + s

    for j in range(CHUNK // 16):
        ones_v[pl.ds(j * 16, 16)] = jnp.full((16,), 1.0, jnp.float32)

    def zrow(r, _):
        zer_v[pl.ds(r * 16, 16)] = jnp.zeros((16,), jnp.float32)
        return 0
    lax.fori_loop(0, RPT // 16, zrow, 0)
    pltpu.sync_copy(zer_v, deg_sh.at[pl.ds(s * RPT, RPT)])
    plsc.subcore_barrier()

    pltpu.sync_copy(dst_hbm.at[pl.ds(wid * NCH, NCH)], dstidx_v)

    def step(j, _):
        pltpu.sync_copy(ones_v, deg_sh.at[dstidx_v.at[j]], add=True)
        return 0
    lax.fori_loop(0, NCH, step, 0)
    plsc.subcore_barrier()

    pltpu.sync_copy(deg_sh.at[pl.ds(s * RPT, RPT)],
                    out_hbm.at[pl.ds(c * NP + s * RPT, RPT)])


_deg_call = functools.partial(
    pl.kernel, _deg_body,
    out_type=jax.ShapeDtypeStruct((NC * NP,), jnp.float32),
    mesh=_mesh,
    scratch_types=[
        pltpu.VMEM((NCH, CHUNK), jnp.int32),
        pltpu.VMEM((CHUNK,), jnp.float32),
        pltpu.VMEM((RPT,), jnp.float32),
        pltpu.VMEM_SHARED((NP,), jnp.float32),
    ],
)()


def _msg_body(g_hbm, src_hbm, dst_hbm, out_hbm,
              srcidx_v, dstidx_v, rows_v, agg_sh, zsem, *gsems):
    c = lax.axis_index("c")
    s = lax.axis_index("s")
    wid = c * NS + s

    def zrow(r, _):
        for j in range(D // 16):
            rows_v[0, r, pl.ds(j * 16, 16)] = jnp.zeros((16,), jnp.float32)
        return 0
    lax.fori_loop(0, CHUNK, zrow, 0)
    zcps = [
        pltpu.async_copy(
            rows_v.at[0], agg_sh.at[pl.ds(s * RPT + k * CHUNK, CHUNK)], zsem)
        for k in range(RPT // CHUNK)
    ]
    pltpu.sync_copy(src_hbm.at[pl.ds(wid * NCH, NCH)], srcidx_v)
    for zc in zcps:
        zc.wait()
    plsc.subcore_barrier()

    for b in range(NBUF):
        pltpu.async_copy(g_hbm.at[srcidx_v.at[b]], rows_v.at[b], gsems[b])

    for q in range(NQ):
        pltpu.sync_copy(dst_hbm.at[pl.ds(wid * NCH + q * QCH, QCH)], dstidx_v)

        def steady(o, _, q=q):
            for b in range(NBUF):
                j = q * QCH + o * NBUF + b
                pltpu.make_async_copy(
                    g_hbm.at[srcidx_v.at[j]], rows_v.at[b], gsems[b]).wait()
                pltpu.sync_copy(
                    rows_v.at[b], agg_sh.at[dstidx_v.at[j - q * QCH]], add=True)
                pltpu.async_copy(
                    g_hbm.at[srcidx_v.at[j + NBUF]], rows_v.at[b], gsems[b])
            return 0

        nout = QCH // NBUF if q < NQ - 1 else QCH // NBUF - 1
        lax.fori_loop(0, nout, steady, 0)

    for b in range(NBUF):
        j = NCH - NBUF + b
        pltpu.make_async_copy(
            g_hbm.at[srcidx_v.at[j]], rows_v.at[b], gsems[b]).wait()
        pltpu.sync_copy(
            rows_v.at[b], agg_sh.at[dstidx_v.at[j - (NQ - 1) * QCH]], add=True)
    plsc.subcore_barrier()

    pltpu.sync_copy(agg_sh.at[pl.ds(s * RPT, RPT)],
                    out_hbm.at[c].at[pl.ds(s * RPT, RPT)])


_msg_call = functools.partial(
    pl.kernel, _msg_body,
    out_type=jax.ShapeDtypeStruct((NC, NP, D), jnp.float32),
    mesh=_mesh,
    scratch_types=[
        pltpu.VMEM((NCH, CHUNK), jnp.int32),
        pltpu.VMEM((QCH, CHUNK), jnp.int32),
        pltpu.VMEM((NBUF, CHUNK, D), jnp.float32),
        pltpu.VMEM_SHARED((NP, D), jnp.float32),
        pltpu.SemaphoreType.DMA,
    ] + [pltpu.SemaphoreType.DMA] * NBUF,
)()


def _dis_body(dp_ref, o_ref):
    dsum = dp_ref[0] + dp_ref[1] + 1.0
    o_ref[...] = lax.rsqrt(jnp.maximum(dsum, 1.0))


def _dis_call(deg_parts):
    return pl.pallas_call(
        _dis_body,
        out_shape=jax.ShapeDtypeStruct((NP // D, D), jnp.float32),
    )(deg_parts)


def _mm_body(z_ref, w_ref, dis_ref, o_ref):
    acc = jnp.dot(z_ref[...], w_ref[...], preferred_element_type=jnp.float32)
    o_ref[...] = acc * dis_ref[...]


def _mm_scale(z, w, dis):
    return pl.pallas_call(
        _mm_body,
        grid=(NP // BLK,),
        in_specs=[
            pl.BlockSpec((BLK, D), lambda i: (i, 0)),
            pl.BlockSpec((D, D), lambda i: (0, 0)),
            pl.BlockSpec((BLK, 1), lambda i: (i, 0)),
        ],
        out_specs=pl.BlockSpec((BLK, D), lambda i: (i, 0)),
        out_shape=jax.ShapeDtypeStruct((NP, D), jnp.float32),
    )(z, w, dis)


def _comb_body_plain(p_ref, g_ref, dis_ref, b_ref, o_ref, *, relu):
    v = (p_ref[0] + p_ref[1] + g_ref[...]) * dis_ref[...] + b_ref[...]
    o_ref[...] = jnp.maximum(v, 0.0) if relu else v


def _comb_body_skip(p_ref, g_ref, dis_ref, b_ref, skip_ref, o_ref):
    v = (p_ref[0] + p_ref[1] + g_ref[...]) * dis_ref[...] + b_ref[...]
    o_ref[...] = jnp.maximum(skip_ref[...] + v, 0.0)


def _combine(parts, g, dis, b, skip=None, relu=False):
    in_specs = [
        pl.BlockSpec((NC, BLK, D), lambda i: (0, i, 0)),
        pl.BlockSpec((BLK, D), lambda i: (i, 0)),
        pl.BlockSpec((BLK, 1), lambda i: (i, 0)),
        pl.BlockSpec((1, D), lambda i: (0, 0)),
    ]
    args = [parts, g, dis, b.reshape(1, D)]
    if skip is None:
        body = functools.partial(_comb_body_plain, relu=relu)
    else:
        body = _comb_body_skip
        in_specs.append(pl.BlockSpec((BLK, D), lambda i: (i, 0)))
        args.append(skip)
    return pl.pallas_call(
        body,
        grid=(NP // BLK,),
        in_specs=in_specs,
        out_specs=pl.BlockSpec((BLK, D), lambda i: (i, 0)),
        out_shape=jax.ShapeDtypeStruct((NP, D), jnp.float32),
    )(*args)


def kernel(x, edge_index, W0, b0, W1, b1, W2, b2):
    src = edge_index[0]
    dst = edge_index[1]
    pad_e = EPAD - E
    src_p = jnp.concatenate(
        [src, jnp.zeros((pad_e,), jnp.int32)]).reshape(NW * NCH, CHUNK)
    dst_p = jnp.concatenate(
        [dst, jnp.full((pad_e,), N, jnp.int32)]).reshape(NW * NCH, CHUNK)
    x_p = jnp.concatenate([x, jnp.zeros((NP - N, D), x.dtype)])

    deg_parts = _deg_call(dst_p)
    dis = _dis_call(deg_parts.reshape(NC, NP // D, D)).reshape(NP, 1)

    g0 = _mm_scale(x_p, W0, dis)
    p0 = _msg_call(g0, src_p, dst_p)
    a0 = _combine(p0, g0, dis, b0, relu=True)

    g1 = _mm_scale(a0, W1, dis)
    p1 = _msg_call(g1, src_p, dst_p)
    a1 = _combine(p1, g1, dis, b1, skip=a0)

    g2 = _mm_scale(a1, W2, dis)
    p2 = _msg_call(g2, src_p, dst_p)
    out = _combine(p2, g2, dis, b2, relu=False)
    return out[:N]
